# baseline (speedup 1.0000x reference)
import jax
import jax.numpy as jnp
from jax import lax
from jax.experimental import pallas as pl
from jax.experimental.pallas import tpu as pltpu

_NBUF = 4
_XSPLIT = 4
_VSPLIT = 2


def _split_copy(src, dst, sem, parts):
    step = src.shape[0] // parts
    return [pltpu.make_async_copy(src.at[pl.ds(k * step, step)],
                                  dst.at[pl.ds(k * step, step)], sem)
            for k in range(parts)]


def _unpool_body(v_hbm, x_hbm, o_hbm, xbuf, vbuf, lsem_x, lsem_v, ssem):
    i = pl.program_id(0)
    n = pl.num_programs(0)
    slot = lax.rem(i, _NBUF)
    nslot = lax.rem(i + 1, _NBUF)

    def load(b, s):
        for c in _split_copy(x_hbm.at[b], xbuf.at[s], lsem_x.at[s], _XSPLIT):
            c.start()
        for c in _split_copy(v_hbm.at[b], vbuf.at[s], lsem_v.at[s], _VSPLIT):
            c.start()

    @pl.when(i == 0)
    def _():
        load(0, 0)

    @pl.when(jnp.logical_and(i + 1 < n, i + 1 >= _NBUF))
    def _():
        for c in _split_copy(xbuf.at[nslot], o_hbm.at[i + 1 - _NBUF],
                             ssem.at[nslot], _XSPLIT):
            c.wait()

    @pl.when(i + 1 < n)
    def _():
        load(i + 1, nslot)

    for c in _split_copy(x_hbm.at[i], xbuf.at[slot], lsem_x.at[slot], _XSPLIT):
        c.wait()
    for c in _split_copy(v_hbm.at[i], vbuf.at[slot], lsem_v.at[slot], _VSPLIT):
        c.wait()

    x = xbuf[slot]
    v2 = vbuf[slot]
    h, w2, c2 = x.shape

    wx = jnp.maximum(x, pltpu.roll(x, c2 // 2, axis=2))
    wr = wx.reshape(h // 2, 2, w2, c2)
    m = jnp.maximum(wr[:, 0], wr[:, 1])

    xr = x.reshape(h // 2, 2, w2, c2)
    oe = jnp.where(xr[:, 0] == m, v2, 0.0)
    oo = jnp.where(xr[:, 1] == m, v2, 0.0)
    xbuf[slot] = jnp.stack([oe, oo], axis=1).reshape(h, w2, c2)

    for c in _split_copy(xbuf.at[slot], o_hbm.at[i], ssem.at[slot], _XSPLIT):
        c.start()

    @pl.when(i == n - 1)
    def _():
        for s in range(_NBUF):
            for c in _split_copy(xbuf.at[s], o_hbm.at[0], ssem.at[s],
                                 _XSPLIT):
                c.wait()


def kernel(input_tensor, pool_input):
    B, H, W, C = pool_input.shape
    Ho, Wo = H // 2, W // 2
    x2 = pool_input.reshape(B, H, Wo, 2 * C)
    v2 = jnp.concatenate([input_tensor, input_tensor], axis=-1)

    out = pl.pallas_call(
        _unpool_body,
        grid=(B,),
        in_specs=[
            pl.BlockSpec(memory_space=pl.ANY),
            pl.BlockSpec(memory_space=pl.ANY),
        ],
        out_specs=pl.BlockSpec(memory_space=pl.ANY),
        out_shape=jax.ShapeDtypeStruct((B, H, Wo, 2 * C), pool_input.dtype),
        scratch_shapes=[
            pltpu.VMEM((_NBUF, H, Wo, 2 * C), pool_input.dtype),
            pltpu.VMEM((_NBUF, Ho, Wo, 2 * C), pool_input.dtype),
            pltpu.SemaphoreType.DMA((_NBUF,)),
            pltpu.SemaphoreType.DMA((_NBUF,)),
            pltpu.SemaphoreType.DMA((_NBUF,)),
        ],
        compiler_params=pltpu.CompilerParams(
            dimension_semantics=("arbitrary",),
        ),
    )(v2, x2)
    return out.reshape(B, H, W, C)


# speedup vs baseline: 1.5998x; 1.5998x over previous
import jax
import jax.numpy as jnp
from jax import lax
from jax.experimental import pallas as pl
from jax.experimental.pallas import tpu as pltpu

_HB = 32


def _unpool_body(vf_ref, x_ref, out_ref):
    x = x_ref[0]
    vf = vf_ref[0]

    hb, c, w = x.shape

    wi = lax.broadcasted_iota(jnp.int32, x.shape, 2)
    nb = jnp.where((wi & 1) == 0,
                   pltpu.roll(x, w - 1, axis=2),
                   pltpu.roll(x, 1, axis=2))
    wx = jnp.maximum(x, nb)

    wr = wx.reshape(hb // 2, 2, c, w)
    m = jnp.maximum(wr[:, 0], wr[:, 1])

    xr = x.reshape(hb // 2, 2, c, w)
    oe = jnp.where(xr[:, 0] == m, vf, 0.0)
    oo = jnp.where(xr[:, 1] == m, vf, 0.0)
    out_ref[0] = jnp.stack([oe, oo], axis=1).reshape(hb, c, w)


def kernel(input_tensor, pool_input):
    B, H, W, C = pool_input.shape
    Ho, Wo = H // 2, W // 2
    nh = H // _HB

    xt = jnp.transpose(pool_input, (0, 1, 3, 2))
    vf = jnp.repeat(jnp.transpose(input_tensor, (0, 1, 3, 2)), 2, axis=3)

    out = pl.pallas_call(
        _unpool_body,
        grid=(B, nh),
        in_specs=[
            pl.BlockSpec((1, _HB // 2, C, W), lambda b, h: (b, h, 0, 0)),
            pl.BlockSpec((1, _HB, C, W), lambda b, h: (b, h, 0, 0)),
        ],
        out_specs=pl.BlockSpec((1, _HB, C, W), lambda b, h: (b, h, 0, 0)),
        out_shape=jax.ShapeDtypeStruct((B, H, C, W), pool_input.dtype),
        compiler_params=pltpu.CompilerParams(
            dimension_semantics=("parallel", "arbitrary"),
        ),
    )(vf, xt)
    return jnp.transpose(out, (0, 1, 3, 2))
